# baseline (speedup 1.0000x reference)
import functools

import jax
import jax.numpy as jnp
from jax import lax
from jax.experimental import pallas as pl
from jax.experimental.pallas import tpu as pltpu
from jax.experimental.pallas import tpu_sc as plsc

NUM_NODES = 10000
NODES_PAD = 10240
TIME_STEPS = 64
LATENT_DIM = 256
TEMPERATURE = 0.5
BATCH = 16

NODE_BLK = 5000
N_NODE_BLKS = NUM_NODES // NODE_BLK

NC1 = 64
NPC1 = NUM_NODES // NC1
NC2 = 8
NPC2 = 64 // NC2

HALF = NUM_NODES // 2
OUTER = 20
OROWS = 256
KSUB = OROWS // 64
LASTG = HALF - OROWS
ACC_UNITS = 2 * (BATCH * NC1) + 256


def _emb_var_body(x_ref, w_ref, b_ref, emb_ref, emb64_ref, var_ref):
    xb = x_ref[0]
    w = w_ref[...]
    emb = lax.dot_general(xb, w, (((1,), (1,)), ((), ())),
                          preferred_element_type=jnp.float32)
    emb = emb + b_ref[...]
    emb_ref[0] = emb
    emb64_ref[0] = emb[:, :TIME_STEPS]
    s = jnp.sum(emb, axis=1)
    ss = jnp.sum(emb * emb, axis=1)
    n = jnp.float32(LATENT_DIM)
    var_ref[0, 0, 0] = (ss - s * s / n) / (n - 1.0)


def _emb_and_var(x, W, b):
    B, N, T = x.shape
    L = W.shape[0]
    emb, emb64, var = pl.pallas_call(
        _emb_var_body,
        grid=(B, N_NODE_BLKS),
        in_specs=[
            pl.BlockSpec((1, NODE_BLK, T), lambda i, j: (i, j, 0)),
            pl.BlockSpec((L, T), lambda i, j: (0, 0)),
            pl.BlockSpec((1, L), lambda i, j: (0, 0)),
        ],
        out_specs=[
            pl.BlockSpec((1, NODE_BLK, L), lambda i, j: (i, j, 0)),
            pl.BlockSpec((1, NODE_BLK, TIME_STEPS), lambda i, j: (i, j, 0)),
            pl.BlockSpec((1, 1, 1, NODE_BLK), lambda i, j: (i, j, 0, 0)),
        ],
        out_shape=[
            jax.ShapeDtypeStruct((B, N, L), jnp.float32),
            jax.ShapeDtypeStruct((B, N, TIME_STEPS), jnp.float32),
            jax.ShapeDtypeStruct((B, N_NODE_BLKS, 1, NODE_BLK), jnp.float32),
        ],
        compiler_params=pltpu.CompilerParams(
            dimension_semantics=("parallel", "parallel")),
    )(x, W, b.reshape(1, L))
    return emb, emb64, var.reshape(B, N)


def _thresh_body(var_ref, ids_ref, cnt_ref):
    v = var_ref[0]
    bits = lax.bitcast_convert_type(v, jnp.int32)
    mono = jnp.where(bits < 0, bits ^ jnp.int32(0x7FFFFFFF), bits)
    key = jnp.maximum(mono, jnp.int32(-1)) + 1
    lane = lax.broadcasted_iota(jnp.int32, (1, NODES_PAD), 1)
    key = jnp.where(lane < NUM_NODES, key, 0)

    r = lax.broadcasted_iota(jnp.int32, (NC1, 1), 0) * NPC1
    rf = r.astype(jnp.float32)
    dcls = lax.broadcasted_iota(jnp.int32, (NC1, 1), 0)
    drow = lax.broadcasted_iota(jnp.int32, (NC1, NC1), 1)
    utri = (lax.broadcasted_iota(jnp.int32, (NC1, NC1), 0)
            >= drow).astype(jnp.float32)
    P = jnp.zeros((NC1, 1), jnp.int32)
    a = jnp.zeros((NC1, 1), jnp.float32)
    for s in (30, 24, 18, 12, 6, 0):
        pref = lax.shift_right_logical(key, min(s + 6, 31))
        cand = (pref == P).astype(jnp.float32)
        dig = lax.shift_right_logical(key, s) & 63
        donehot = (dig == dcls).astype(jnp.float32)
        c = lax.dot_general(cand, donehot, (((1,), (1,)), ((), ())),
                            preferred_element_type=jnp.float32)
        st = lax.dot_general(c, utri, (((1,), (0,)), ((), ())),
                             preferred_element_type=jnp.float32)
        t = a + st
        ok = (t >= rf).astype(jnp.int32)
        dstar = jnp.sum(ok, axis=1, keepdims=True) - 1
        sel = (drow == dstar).astype(jnp.float32)
        a = jnp.sum((t - c) * sel, axis=1, keepdims=True)
        P = P * 64 + dstar
    thr = jnp.where(r >= 1, P, jnp.int32(0x7FFFFFFF))

    in_top = (key >= thr).astype(jnp.int32)
    bucket = (NC1 - 1) - jnp.sum(in_top, axis=0, keepdims=True)

    cnt_top = jnp.sum(in_top[:, :NUM_NODES], axis=1, keepdims=True)
    nxt = jnp.concatenate(
        [cnt_top[1:], jnp.full((1, 1), NUM_NODES, jnp.int32)], axis=0)
    cnt_ref[0] = (nxt - cnt_top).astype(jnp.float32)

    bi = pl.program_id(0)
    units = jnp.repeat((bucket + NC1 * bi) * 2, 2, axis=1)
    off = lax.broadcasted_iota(jnp.int32, (1, 2 * OROWS), 1)
    units_par = off & 1
    dummy = 2 * BATCH * NC1 + ((off + 16 * bi) & 255)
    halves = []
    for ci in range(2):
        chunks = []
        for j in range(OUTER):
            g = min(j * OROWS, LASTG)
            st0 = 2 * (ci * HALF + g)
            sl = lax.slice(units, (0, st0), (1, st0 + 2 * OROWS)) + units_par
            if j == OUTER - 1:
                sl = jnp.where(off >= 2 * ((OUTER - 1) * OROWS - LASTG),
                               sl, dummy)
            chunks.append(sl.reshape(KSUB, 128))
        halves.append(jnp.stack(chunks))
    ids_ref[0] = jnp.stack(halves)


def _thresholds(var_pad):
    return pl.pallas_call(
        _thresh_body,
        grid=(BATCH,),
        in_specs=[pl.BlockSpec((1, 1, NODES_PAD), lambda i: (i, 0, 0))],
        out_specs=[
            pl.BlockSpec((1, 2, OUTER, KSUB, 128), lambda i: (i, 0, 0, 0, 0)),
            pl.BlockSpec((1, NC1, 1), lambda i: (i, 0, 0)),
        ],
        out_shape=[
            jax.ShapeDtypeStruct((BATCH, 2, OUTER, KSUB, 128), jnp.int32),
            jax.ShapeDtypeStruct((BATCH, NC1, 1), jnp.float32),
        ],
    )(var_pad)


def _sc_body(emb64, ids, zz, out, rowbuf,
             idb0, idb1, idb2, idb3, acc):
    ci = lax.axis_index("c")
    s = lax.axis_index("s")
    idbufs = (idb0, idb1, idb2, idb3)
    zrows = (ACC_UNITS // 2) // BATCH
    pltpu.sync_copy(zz, acc.at[pl.ds(s * zrows, zrows)])
    plsc.subcore_barrier()

    def chunk(j, carry):
        g = jnp.minimum(j * OROWS, LASTG)
        nb = ci * HALF + g
        pltpu.sync_copy(emb64.at[s, pl.ds(nb, OROWS)],
                        rowbuf.at[pl.ds(0, OROWS)])
        for k in range(KSUB):
            pltpu.sync_copy(ids.at[s, ci, j, k], idbufs[k])
        for k in range(KSUB):
            pltpu.sync_copy(rowbuf.at[pl.ds(k * 64, 128)],
                            acc.at[idbufs[k]], add=True)
        return carry

    lax.fori_loop(0, OUTER, chunk, 0)
    plsc.subcore_barrier()
    pltpu.sync_copy(acc.at[pl.ds(s * zrows, zrows)],
                    out.at[ci, pl.ds(s * zrows, zrows)])


def _sc_segment_sums(emb64, ids):
    zrows = (ACC_UNITS // 2) // BATCH
    mesh = plsc.VectorSubcoreMesh(core_axis_name="c", subcore_axis_name="s")
    fn = functools.partial(
        pl.kernel,
        out_type=jax.ShapeDtypeStruct(
            (2, ACC_UNITS // 2, TIME_STEPS), jnp.float32),
        mesh=mesh,
        scratch_types=(
            [pltpu.VMEM((OROWS + 64, TIME_STEPS), jnp.float32)]
            + [pltpu.VMEM((128,), jnp.int32) for _ in range(KSUB)]
            + [pltpu.VMEM_SHARED((ACC_UNITS // 2, TIME_STEPS), jnp.float32)]
        ),
    )(_sc_body)
    zz = jnp.zeros((zrows, TIME_STEPS), jnp.float32)
    return fn(emb64, ids, zz)


def _final_body(sums_ref, cnt_ref, loss_ref):
    arr = sums_ref[...]
    tot = arr[0, :BATCH * NC1] + arr[1, :BATCH * NC1]
    aggs = []
    for bi in range(BATCH):
        sb = tot[bi * NC1:(bi + 1) * NC1]
        cb = cnt_ref[bi]
        agg1 = sb / jnp.maximum(cb, 1.0)
        s2 = jnp.sum(agg1, axis=1, keepdims=True)
        ss2 = jnp.sum(agg1 * agg1, axis=1, keepdims=True)
        n2 = jnp.float32(TIME_STEPS)
        v2 = (ss2 - s2 * s2 / n2) / (n2 - 1.0)
        eye = (lax.broadcasted_iota(jnp.int32, (NC1, NC1), 0)
               == lax.broadcasted_iota(jnp.int32, (NC1, NC1), 1))
        v2r = jnp.sum(jnp.where(eye, v2, 0.0), axis=0, keepdims=True)
        irow = lax.broadcasted_iota(jnp.int32, (NC1, NC1), 0)
        jcol = lax.broadcasted_iota(jnp.int32, (NC1, NC1), 1)
        ahead = (v2 > v2r) | ((v2 == v2r) & (irow < jcol))
        rank2 = jnp.sum(ahead.astype(jnp.int32), axis=0, keepdims=True)
        bucket2 = rank2 // NPC2
        cls2 = lax.broadcasted_iota(jnp.int32, (NC2, 1), 0)
        onehot2 = (bucket2 == cls2).astype(jnp.float32)
        agg2 = lax.dot_general(onehot2, agg1, (((1,), (0,)), ((), ())),
                               preferred_element_type=jnp.float32)
        aggs.append(agg2 * jnp.float32(1.0 / NPC2))
    e = jnp.concatenate(aggs, axis=0)

    nsq = jnp.sum(e * e, axis=1, keepdims=True)
    n1 = jnp.sqrt(nsq)
    m = BATCH * NC2
    eyem = (lax.broadcasted_iota(jnp.int32, (m, m), 0)
            == lax.broadcasted_iota(jnp.int32, (m, m), 1))
    n1r = jnp.sum(jnp.where(eyem, n1, 0.0), axis=0, keepdims=True)
    dots = lax.dot_general(e, e, (((1,), (1,)), ((), ())),
                           preferred_element_type=jnp.float32)
    sim = dots / jnp.maximum(n1 * n1r, 1e-8)
    logits = sim * jnp.float32(1.0 / TEMPERATURE)
    lab = eyem.astype(jnp.float32)
    loss_mat = (jnp.maximum(logits, 0.0) - logits * lab
                + jnp.log1p(jnp.exp(-jnp.abs(logits))))
    totl = jnp.sum(jnp.sum(loss_mat, axis=1, keepdims=True),
                   axis=0, keepdims=True)
    loss_ref[...] = totl * jnp.float32(1.0 / (m * m))


def _final(sums, counts):
    out = pl.pallas_call(
        _final_body,
        out_shape=jax.ShapeDtypeStruct((1, 1), jnp.float32),
    )(sums, counts)
    return out.reshape(())


def kernel(x, W, b):
    emb, emb64, var1 = _emb_and_var(x, W, b)
    var_pad = jnp.pad(var1, ((0, 0), (0, NODES_PAD - NUM_NODES)),
                      constant_values=-1.0).reshape(BATCH, 1, NODES_PAD)
    ids, counts = _thresholds(var_pad)
    sums = _sc_segment_sums(emb64, ids)
    return (_final(sums, counts), emb)


# speedup vs baseline: 2.0803x; 2.0803x over previous
import jax
import jax.numpy as jnp
from jax import lax
from jax.experimental import pallas as pl
from jax.experimental.pallas import tpu as pltpu

NUM_NODES = 10000
NODES_PAD = 10240
TIME_STEPS = 64
LATENT_DIM = 256
TEMPERATURE = 0.5
BATCH = 16

NODE_BLK = 5000
N_NODE_BLKS = NUM_NODES // NODE_BLK

NC1 = 64
NPC1 = NUM_NODES // NC1
NC2 = 8
NPC2 = 64 // NC2


def _emb_var_body(x_ref, w_ref, b_ref, emb_ref, var_ref):
    xb = x_ref[0]
    w = w_ref[...]
    emb = lax.dot_general(xb, w, (((1,), (1,)), ((), ())),
                          preferred_element_type=jnp.float32)
    emb = emb + b_ref[...]
    emb_ref[0] = emb
    s = jnp.sum(emb, axis=1)
    ss = jnp.sum(emb * emb, axis=1)
    n = jnp.float32(LATENT_DIM)
    var_ref[0, 0, 0] = (ss - s * s / n) / (n - 1.0)


def _emb_and_var(x, W, b):
    B, N, T = x.shape
    L = W.shape[0]
    emb, var = pl.pallas_call(
        _emb_var_body,
        grid=(B, N_NODE_BLKS),
        in_specs=[
            pl.BlockSpec((1, NODE_BLK, T), lambda i, j: (i, j, 0)),
            pl.BlockSpec((L, T), lambda i, j: (0, 0)),
            pl.BlockSpec((1, L), lambda i, j: (0, 0)),
        ],
        out_specs=[
            pl.BlockSpec((1, NODE_BLK, L), lambda i, j: (i, j, 0)),
            pl.BlockSpec((1, 1, 1, NODE_BLK), lambda i, j: (i, j, 0, 0)),
        ],
        out_shape=[
            jax.ShapeDtypeStruct((B, N, L), jnp.float32),
            jax.ShapeDtypeStruct((B, N_NODE_BLKS, 1, NODE_BLK), jnp.float32),
        ],
        compiler_params=pltpu.CompilerParams(
            dimension_semantics=("parallel", "parallel")),
    )(x, W, b.reshape(1, L))
    return emb, var.reshape(B, N)


def _agg_body(var_ref, emb_ref, agg2_ref):
    v = var_ref[0]
    bits = lax.bitcast_convert_type(v, jnp.int32)
    mono = jnp.where(bits < 0, bits ^ jnp.int32(0x7FFFFFFF), bits)
    key = jnp.maximum(mono, jnp.int32(-1)) + 1
    lane = lax.broadcasted_iota(jnp.int32, (1, NODES_PAD), 1)
    key = jnp.where(lane < NUM_NODES, key, 0)

    r = lax.broadcasted_iota(jnp.int32, (NC1, 1), 0) * NPC1
    rf = r.astype(jnp.float32)
    dcls = lax.broadcasted_iota(jnp.int32, (NC1, 1), 0)
    drow = lax.broadcasted_iota(jnp.int32, (NC1, NC1), 1)
    utri = (lax.broadcasted_iota(jnp.int32, (NC1, NC1), 0)
            >= drow).astype(jnp.float32)
    P = jnp.zeros((NC1, 1), jnp.int32)
    a = jnp.zeros((NC1, 1), jnp.float32)
    for s in (30, 24, 18, 12, 6, 0):
        pref = lax.shift_right_logical(key, min(s + 6, 31))
        cand = (pref == P).astype(jnp.float32)
        dig = lax.shift_right_logical(key, s) & 63
        donehot = (dig == dcls).astype(jnp.float32)
        c = lax.dot_general(cand, donehot, (((1,), (1,)), ((), ())),
                            preferred_element_type=jnp.float32)
        st = lax.dot_general(c, utri, (((1,), (0,)), ((), ())),
                             preferred_element_type=jnp.float32)
        t = a + st
        ok = (t >= rf).astype(jnp.int32)
        dstar = jnp.sum(ok, axis=1, keepdims=True) - 1
        sel = (drow == dstar).astype(jnp.float32)
        a = jnp.sum((t - c) * sel, axis=1, keepdims=True)
        P = P * 64 + dstar
    thr = jnp.where(r >= 1, P, jnp.int32(0x7FFFFFFF))

    in_top = (key >= thr).astype(jnp.int32)
    bucket = (NC1 - 1) - jnp.sum(in_top, axis=0, keepdims=True)
    cls = lax.broadcasted_iota(jnp.int32, (NC1, 1), 0)
    onehot = (bucket == cls).astype(jnp.float32)
    onehot = onehot[:, :NUM_NODES]
    counts = jnp.sum(onehot, axis=1, keepdims=True)
    e64 = emb_ref[0][:, :TIME_STEPS]
    sums = lax.dot_general(onehot, e64, (((1,), (0,)), ((), ())),
                           preferred_element_type=jnp.float32)
    agg1 = sums / jnp.maximum(counts, 1.0)

    s2 = jnp.sum(agg1, axis=1, keepdims=True)
    ss2 = jnp.sum(agg1 * agg1, axis=1, keepdims=True)
    n2 = jnp.float32(TIME_STEPS)
    v2 = (ss2 - s2 * s2 / n2) / (n2 - 1.0)
    eye = (lax.broadcasted_iota(jnp.int32, (NC1, NC1), 0)
           == lax.broadcasted_iota(jnp.int32, (NC1, NC1), 1))
    v2r = jnp.sum(jnp.where(eye, v2, 0.0), axis=0, keepdims=True)
    irow = lax.broadcasted_iota(jnp.int32, (NC1, NC1), 0)
    jcol = lax.broadcasted_iota(jnp.int32, (NC1, NC1), 1)
    ahead = (v2 > v2r) | ((v2 == v2r) & (irow < jcol))
    rank2 = jnp.sum(ahead.astype(jnp.int32), axis=0, keepdims=True)
    bucket2 = rank2 // NPC2
    cls2 = lax.broadcasted_iota(jnp.int32, (NC2, 1), 0)
    onehot2 = (bucket2 == cls2).astype(jnp.float32)
    agg2 = lax.dot_general(onehot2, agg1, (((1,), (0,)), ((), ())),
                           preferred_element_type=jnp.float32)
    agg2_ref[0] = agg2 * jnp.float32(1.0 / NPC2)


def _aggregate2(var_pad, emb):
    B = emb.shape[0]
    return pl.pallas_call(
        _agg_body,
        grid=(B,),
        in_specs=[
            pl.BlockSpec((1, 1, NODES_PAD), lambda i: (i, 0, 0)),
            pl.BlockSpec((1, NUM_NODES, 128), lambda i: (i, 0, 0)),
        ],
        out_specs=pl.BlockSpec((1, NC2, TIME_STEPS), lambda i: (i, 0, 0)),
        out_shape=jax.ShapeDtypeStruct((B, NC2, TIME_STEPS), jnp.float32),
    )(var_pad, emb)


def _loss_body(agg2_ref, loss_ref):
    e = agg2_ref[...].reshape(BATCH * NC2, TIME_STEPS)
    nsq = jnp.sum(e * e, axis=1, keepdims=True)
    n1 = jnp.sqrt(nsq)
    m = BATCH * NC2
    eye = (lax.broadcasted_iota(jnp.int32, (m, m), 0)
           == lax.broadcasted_iota(jnp.int32, (m, m), 1))
    n1r = jnp.sum(jnp.where(eye, n1, 0.0), axis=0, keepdims=True)
    dots = lax.dot_general(e, e, (((1,), (1,)), ((), ())),
                           preferred_element_type=jnp.float32)
    sim = dots / jnp.maximum(n1 * n1r, 1e-8)
    logits = sim * jnp.float32(1.0 / TEMPERATURE)
    lab = eye.astype(jnp.float32)
    loss_mat = (jnp.maximum(logits, 0.0) - logits * lab
                + jnp.log1p(jnp.exp(-jnp.abs(logits))))
    tot = jnp.sum(jnp.sum(loss_mat, axis=1, keepdims=True),
                  axis=0, keepdims=True)
    loss_ref[...] = tot * jnp.float32(1.0 / (m * m))


def _loss(agg2):
    out = pl.pallas_call(
        _loss_body,
        out_shape=jax.ShapeDtypeStruct((1, 1), jnp.float32),
    )(agg2)
    return out.reshape(())


def kernel(x, W, b):
    emb, var1 = _emb_and_var(x, W, b)
    var_pad = jnp.pad(var1, ((0, 0), (0, NODES_PAD - NUM_NODES)),
                      constant_values=-1.0).reshape(BATCH, 1, NODES_PAD)
    agg2 = _aggregate2(var_pad, emb)
    return (_loss(agg2), emb)


# speedup vs baseline: 2.0837x; 1.0016x over previous
import jax
import jax.numpy as jnp
from jax import lax
from jax.experimental import pallas as pl
from jax.experimental.pallas import tpu as pltpu

NUM_NODES = 10000
NODES_PAD = 10240
TIME_STEPS = 64
LATENT_DIM = 256
TEMPERATURE = 0.5
BATCH = 16

NODE_BLK = 10000
N_NODE_BLKS = NUM_NODES // NODE_BLK

NC1 = 64
NPC1 = NUM_NODES // NC1
NC2 = 8
NPC2 = 64 // NC2


def _emb_var_body(x_ref, w_ref, b_ref, emb_ref, var_ref):
    xb = x_ref[0]
    w = w_ref[...]
    emb = lax.dot_general(xb, w, (((1,), (1,)), ((), ())),
                          preferred_element_type=jnp.float32)
    emb = emb + b_ref[...]
    emb_ref[0] = emb
    s = jnp.sum(emb, axis=1)
    ss = jnp.sum(emb * emb, axis=1)
    n = jnp.float32(LATENT_DIM)
    var_ref[0, 0, 0] = (ss - s * s / n) / (n - 1.0)


def _emb_and_var(x, W, b):
    B, N, T = x.shape
    L = W.shape[0]
    emb, var = pl.pallas_call(
        _emb_var_body,
        grid=(B, N_NODE_BLKS),
        in_specs=[
            pl.BlockSpec((1, NODE_BLK, T), lambda i, j: (i, j, 0)),
            pl.BlockSpec((L, T), lambda i, j: (0, 0)),
            pl.BlockSpec((1, L), lambda i, j: (0, 0)),
        ],
        out_specs=[
            pl.BlockSpec((1, NODE_BLK, L), lambda i, j: (i, j, 0)),
            pl.BlockSpec((1, 1, 1, NODE_BLK), lambda i, j: (i, j, 0, 0)),
        ],
        out_shape=[
            jax.ShapeDtypeStruct((B, N, L), jnp.float32),
            jax.ShapeDtypeStruct((B, N_NODE_BLKS, 1, NODE_BLK), jnp.float32),
        ],
        compiler_params=pltpu.CompilerParams(
            dimension_semantics=("parallel", "parallel")),
    )(x, W, b.reshape(1, L))
    return emb, var.reshape(B, N)


def _agg_body(var_ref, emb_ref, agg2_ref):
    v = var_ref[0]
    bits = lax.bitcast_convert_type(v, jnp.int32)
    mono = jnp.where(bits < 0, bits ^ jnp.int32(0x7FFFFFFF), bits)
    key = jnp.maximum(mono, jnp.int32(-1)) + 1
    lane = lax.broadcasted_iota(jnp.int32, (1, NODES_PAD), 1)
    key = jnp.where(lane < NUM_NODES, key, 0)

    r = lax.broadcasted_iota(jnp.int32, (NC1, 1), 0) * NPC1
    rf = r.astype(jnp.float32)
    dcls = lax.broadcasted_iota(jnp.int32, (NC1, 1), 0)
    drow = lax.broadcasted_iota(jnp.int32, (NC1, NC1), 1)
    utri = (lax.broadcasted_iota(jnp.int32, (NC1, NC1), 0)
            >= drow).astype(jnp.float32)
    P = jnp.zeros((NC1, 1), jnp.int32)
    a = jnp.zeros((NC1, 1), jnp.float32)
    for s in (30, 24, 18, 12, 6, 0):
        pref = lax.shift_right_logical(key, min(s + 6, 31))
        cand = (pref == P).astype(jnp.float32)
        dig = lax.shift_right_logical(key, s) & 63
        donehot = (dig == dcls).astype(jnp.float32)
        c = lax.dot_general(cand, donehot, (((1,), (1,)), ((), ())),
                            preferred_element_type=jnp.float32)
        st = lax.dot_general(c, utri, (((1,), (0,)), ((), ())),
                             preferred_element_type=jnp.float32)
        t = a + st
        ok = (t >= rf).astype(jnp.int32)
        dstar = jnp.sum(ok, axis=1, keepdims=True) - 1
        sel = (drow == dstar).astype(jnp.float32)
        a = jnp.sum((t - c) * sel, axis=1, keepdims=True)
        P = P * 64 + dstar
    thr = jnp.where(r >= 1, P, jnp.int32(0x7FFFFFFF))

    in_top = (key >= thr).astype(jnp.int32)
    bucket = (NC1 - 1) - jnp.sum(in_top, axis=0, keepdims=True)
    cls = lax.broadcasted_iota(jnp.int32, (NC1, 1), 0)
    onehot = (bucket == cls).astype(jnp.float32)
    onehot = onehot[:, :NUM_NODES]
    counts = jnp.sum(onehot, axis=1, keepdims=True)
    e64 = emb_ref[0][:, :TIME_STEPS]
    sums = lax.dot_general(onehot, e64, (((1,), (0,)), ((), ())),
                           preferred_element_type=jnp.float32)
    agg1 = sums / jnp.maximum(counts, 1.0)

    s2 = jnp.sum(agg1, axis=1, keepdims=True)
    ss2 = jnp.sum(agg1 * agg1, axis=1, keepdims=True)
    n2 = jnp.float32(TIME_STEPS)
    v2 = (ss2 - s2 * s2 / n2) / (n2 - 1.0)
    eye = (lax.broadcasted_iota(jnp.int32, (NC1, NC1), 0)
           == lax.broadcasted_iota(jnp.int32, (NC1, NC1), 1))
    v2r = jnp.sum(jnp.where(eye, v2, 0.0), axis=0, keepdims=True)
    irow = lax.broadcasted_iota(jnp.int32, (NC1, NC1), 0)
    jcol = lax.broadcasted_iota(jnp.int32, (NC1, NC1), 1)
    ahead = (v2 > v2r) | ((v2 == v2r) & (irow < jcol))
    rank2 = jnp.sum(ahead.astype(jnp.int32), axis=0, keepdims=True)
    bucket2 = rank2 // NPC2
    cls2 = lax.broadcasted_iota(jnp.int32, (NC2, 1), 0)
    onehot2 = (bucket2 == cls2).astype(jnp.float32)
    agg2 = lax.dot_general(onehot2, agg1, (((1,), (0,)), ((), ())),
                           preferred_element_type=jnp.float32)
    agg2_ref[0] = agg2 * jnp.float32(1.0 / NPC2)


def _aggregate2(var_pad, emb):
    B = emb.shape[0]
    return pl.pallas_call(
        _agg_body,
        grid=(B,),
        in_specs=[
            pl.BlockSpec((1, 1, NODES_PAD), lambda i: (i, 0, 0)),
            pl.BlockSpec((1, NUM_NODES, 128), lambda i: (i, 0, 0)),
        ],
        out_specs=pl.BlockSpec((1, NC2, TIME_STEPS), lambda i: (i, 0, 0)),
        out_shape=jax.ShapeDtypeStruct((B, NC2, TIME_STEPS), jnp.float32),
    )(var_pad, emb)


def _loss_body(agg2_ref, loss_ref):
    e = agg2_ref[...].reshape(BATCH * NC2, TIME_STEPS)
    nsq = jnp.sum(e * e, axis=1, keepdims=True)
    n1 = jnp.sqrt(nsq)
    m = BATCH * NC2
    eye = (lax.broadcasted_iota(jnp.int32, (m, m), 0)
           == lax.broadcasted_iota(jnp.int32, (m, m), 1))
    n1r = jnp.sum(jnp.where(eye, n1, 0.0), axis=0, keepdims=True)
    dots = lax.dot_general(e, e, (((1,), (1,)), ((), ())),
                           preferred_element_type=jnp.float32)
    sim = dots / jnp.maximum(n1 * n1r, 1e-8)
    logits = sim * jnp.float32(1.0 / TEMPERATURE)
    lab = eye.astype(jnp.float32)
    loss_mat = (jnp.maximum(logits, 0.0) - logits * lab
                + jnp.log1p(jnp.exp(-jnp.abs(logits))))
    tot = jnp.sum(jnp.sum(loss_mat, axis=1, keepdims=True),
                  axis=0, keepdims=True)
    loss_ref[...] = tot * jnp.float32(1.0 / (m * m))


def _loss(agg2):
    out = pl.pallas_call(
        _loss_body,
        out_shape=jax.ShapeDtypeStruct((1, 1), jnp.float32),
    )(agg2)
    return out.reshape(())


def kernel(x, W, b):
    emb, var1 = _emb_and_var(x, W, b)
    var_pad = jnp.pad(var1, ((0, 0), (0, NODES_PAD - NUM_NODES)),
                      constant_values=-1.0).reshape(BATCH, 1, NODES_PAD)
    agg2 = _aggregate2(var_pad, emb)
    return (_loss(agg2), emb)
